# baseline (speedup 1.0000x reference)
from functools import partial

import numpy as np
import jax
import jax.numpy as jnp
from jax.experimental import pallas as pl
from jax.experimental.pallas import tpu as pltpu

_KS0 = np.uint32(0)
_KS1 = np.uint32(42)
_KS2 = np.uint32(0 ^ 42 ^ 0x1BD11BDA)
_ROTS = (13, 15, 26, 6, 17, 29, 16, 24, 13, 15, 26, 6, 17, 29, 16, 24, 13, 15, 26, 6)
_INJ = {
    3: (_KS1, _KS2 + np.uint32(1)),
    7: (_KS2, _KS0 + np.uint32(2)),
    11: (_KS0, _KS1 + np.uint32(3)),
    15: (_KS1, _KS2 + np.uint32(4)),
    19: (_KS2, _KS0 + np.uint32(5)),
}
_TINY = np.float32(np.finfo(np.float32).tiny)
_I32MAX = np.int32(2**31 - 1)


def _rotl(x, r):
    return (x << np.uint32(r)) | (x >> np.uint32(32 - r))


def _threefry_bits(x1):
    x0 = x1
    x1 = _rotl(x1, _ROTS[0]) ^ x0
    for rnd in range(1, 20):
        x0 = x0 + x1
        x1 = _rotl(x1, _ROTS[rnd]) ^ x0
        inj = _INJ.get(rnd)
        if inj is not None:
            a, b = inj
            if a:
                x0 = x0 + a
            if b:
                x1 = x1 + b
    return x0 ^ x1


def _sampler_kernel(logits_ref, out_ref, acc_ref, acck_ref, *,
                    bc, vocab, nblocks):
    j = pl.program_id(0)
    bsz = logits_ref.shape[0]
    cw = 1024
    nchunks = bc // cw
    c0 = j * bc
    lane = jax.lax.broadcasted_iota(jnp.int32, (bsz, cw), 1)
    row = jax.lax.broadcasted_iota(jnp.int32, (bsz, cw), 0)
    base = row * vocab + lane

    @pl.when(j == 0)
    def _():
        acc_ref[...] = jnp.full_like(acc_ref, -jnp.inf)
        acck_ref[...] = jnp.zeros_like(acck_ref)

    pad0 = vocab - (nblocks - 1) * bc
    if pad0 < bc:
        @pl.when(j == nblocks - 1)
        def _():
            logits_ref[:, pad0:] = jnp.full((bsz, bc - pad0), -jnp.inf,
                                            jnp.float32)

    acc = acc_ref[...]
    acck = acck_ref[...]
    for k in range(nchunks):
        off = k * cw
        colc = c0 + off
        chunk = logits_ref[:, off:off + cw]
        x1 = jax.lax.bitcast_convert_type(base + (colc + np.int32(42)), jnp.uint32)
        bits = _threefry_bits(x1)
        u = jax.lax.bitcast_convert_type(
            (bits >> np.uint32(9)) | np.uint32(0x3F800000), jnp.float32) - 1.0
        w = -jnp.log(u)
        pert = chunk - jnp.log(w)
        acck = jnp.where(pert > acc, jnp.int32(j * nchunks + k), acck)
        acc = jnp.maximum(acc, pert)
    acc_ref[...] = acc
    acck_ref[...] = acck

    @pl.when(j == nblocks - 1)
    def _():
        col = acck * cw + lane
        m = jnp.max(acc, axis=1, keepdims=True)
        out_ref[...] = jnp.min(
            jnp.where(acc == m, col, _I32MAX), axis=1, keepdims=True)


def _build(shape, bc):
    bsz, vocab = shape
    nblocks = pl.cdiv(vocab, bc)
    return pl.pallas_call(
        partial(_sampler_kernel, bc=bc, vocab=vocab, nblocks=nblocks),
        grid=(nblocks,),
        in_specs=[pl.BlockSpec((bsz, bc), lambda j: (0, j))],
        out_specs=pl.BlockSpec((bsz, 1), lambda j: (0, 0)),
        out_shape=jax.ShapeDtypeStruct((bsz, 1), jnp.int32),
        scratch_shapes=[pltpu.VMEM((bsz, 1024), jnp.float32),
                        pltpu.VMEM((bsz, 1024), jnp.int32)],
    )


def kernel(logits):
    samples = _build(logits.shape, bc=16384)(logits)
    return samples.astype(jnp.int64)


# speedup vs baseline: 1.0077x; 1.0077x over previous
from functools import partial

import numpy as np
import jax
import jax.numpy as jnp
from jax.experimental import pallas as pl
from jax.experimental.pallas import tpu as pltpu

_KS0 = np.uint32(0)
_KS1 = np.uint32(42)
_KS2 = np.uint32(0 ^ 42 ^ 0x1BD11BDA)
_ROTS = (13, 15, 26, 6, 17, 29, 16, 24, 13, 15, 26, 6, 17, 29, 16, 24, 13, 15, 26, 6)
_INJ = {
    3: (_KS1, _KS2 + np.uint32(1)),
    7: (_KS2, _KS0 + np.uint32(2)),
    11: (_KS0, _KS1 + np.uint32(3)),
    15: (_KS1, _KS2 + np.uint32(4)),
    19: (_KS2, _KS0 + np.uint32(5)),
}
_TINY = np.float32(np.finfo(np.float32).tiny)
_I32MAX = np.int32(2**31 - 1)


def _rotl(x, r):
    return (x << np.uint32(r)) | (x >> np.uint32(32 - r))


def _threefry_bits(x1):
    x0 = x1
    x1 = _rotl(x1, _ROTS[0]) ^ x0
    for rnd in range(1, 20):
        x0 = x0 + x1
        x1 = _rotl(x1, _ROTS[rnd]) ^ x0
        inj = _INJ.get(rnd)
        if inj is not None:
            a, b = inj
            if a:
                x0 = x0 + a
            if b:
                x1 = x1 + b
    return x0 ^ x1


def _sampler_kernel(logits_ref, out_ref, acc_ref, acck_ref, *,
                    bc, vocab, nblocks):
    j = pl.program_id(0)
    bsz = logits_ref.shape[0]
    cw = 512
    nchunks = bc // cw
    c0 = j * bc
    lane = jax.lax.broadcasted_iota(jnp.int32, (bsz, cw), 1)
    row = jax.lax.broadcasted_iota(jnp.int32, (bsz, cw), 0)
    base = row * vocab + lane

    @pl.when(j == 0)
    def _():
        acc_ref[...] = jnp.full_like(acc_ref, -jnp.inf)
        acck_ref[...] = jnp.zeros_like(acck_ref)

    pad0 = vocab - (nblocks - 1) * bc
    if pad0 < bc:
        @pl.when(j == nblocks - 1)
        def _():
            logits_ref[:, pad0:] = jnp.full((bsz, bc - pad0), -jnp.inf,
                                            jnp.float32)

    acc = acc_ref[...]
    acck = acck_ref[...]
    for k in range(nchunks):
        off = k * cw
        colc = c0 + off
        chunk = logits_ref[:, off:off + cw]
        x1 = jax.lax.bitcast_convert_type(base + (colc + np.int32(42)), jnp.uint32)
        bits = _threefry_bits(x1)
        u = jax.lax.bitcast_convert_type(
            (bits >> np.uint32(9)) | np.uint32(0x3F800000), jnp.float32) - 1.0
        w = -jnp.log(u)
        pert = chunk - jnp.log(w)
        acck = jnp.where(pert > acc, jnp.int32(j * nchunks + k), acck)
        acc = jnp.maximum(acc, pert)
    acc_ref[...] = acc
    acck_ref[...] = acck

    @pl.when(j == nblocks - 1)
    def _():
        col = acck * cw + lane
        m = jnp.max(acc, axis=1, keepdims=True)
        out_ref[...] = jnp.min(
            jnp.where(acc == m, col, _I32MAX), axis=1, keepdims=True)


def _build(shape, bc):
    bsz, vocab = shape
    nblocks = pl.cdiv(vocab, bc)
    return pl.pallas_call(
        partial(_sampler_kernel, bc=bc, vocab=vocab, nblocks=nblocks),
        grid=(nblocks,),
        in_specs=[pl.BlockSpec((bsz, bc), lambda j: (0, j))],
        out_specs=pl.BlockSpec((bsz, 1), lambda j: (0, 0)),
        out_shape=jax.ShapeDtypeStruct((bsz, 1), jnp.int32),
        scratch_shapes=[pltpu.VMEM((bsz, 512), jnp.float32),
                        pltpu.VMEM((bsz, 512), jnp.int32)],
    )


def kernel(logits):
    samples = _build(logits.shape, bc=8192)(logits)
    return samples.astype(jnp.int64)


# speedup vs baseline: 1.0093x; 1.0016x over previous
from functools import partial

import numpy as np
import jax
import jax.numpy as jnp
from jax.experimental import pallas as pl
from jax.experimental.pallas import tpu as pltpu

_KS0 = np.uint32(0)
_KS1 = np.uint32(42)
_KS2 = np.uint32(0 ^ 42 ^ 0x1BD11BDA)
_ROTS = (13, 15, 26, 6, 17, 29, 16, 24, 13, 15, 26, 6, 17, 29, 16, 24, 13, 15, 26, 6)
_INJ = {
    3: (_KS1, _KS2 + np.uint32(1)),
    7: (_KS2, _KS0 + np.uint32(2)),
    11: (_KS0, _KS1 + np.uint32(3)),
    15: (_KS1, _KS2 + np.uint32(4)),
    19: (_KS2, _KS0 + np.uint32(5)),
}
_TINY = np.float32(np.finfo(np.float32).tiny)
_I32MAX = np.int32(2**31 - 1)


def _rotl(x, r):
    return (x << np.uint32(r)) | (x >> np.uint32(32 - r))


def _threefry_bits(x1):
    x0 = x1
    x1 = _rotl(x1, _ROTS[0]) ^ x0
    for rnd in range(1, 20):
        x0 = x0 + x1
        x1 = _rotl(x1, _ROTS[rnd]) ^ x0
        inj = _INJ.get(rnd)
        if inj is not None:
            a, b = inj
            if a:
                x0 = x0 + a
            if b:
                x1 = x1 + b
    return x0 ^ x1


def _sampler_kernel(logits_ref, out_ref, acc_ref, acck_ref, *,
                    bc, vocab, nblocks):
    j = pl.program_id(0)
    bsz = logits_ref.shape[0]
    cw = 512
    nchunks = bc // cw
    c0 = j * bc
    lane = jax.lax.broadcasted_iota(jnp.int32, (bsz, cw), 1)
    row = jax.lax.broadcasted_iota(jnp.int32, (bsz, cw), 0)
    base = row * vocab + lane

    @pl.when(j == 0)
    def _():
        acc_ref[...] = jnp.full_like(acc_ref, -jnp.inf)
        acck_ref[...] = jnp.zeros_like(acck_ref)

    pad0 = vocab - (nblocks - 1) * bc
    if pad0 < bc:
        @pl.when(j == nblocks - 1)
        def _():
            logits_ref[:, pad0:] = jnp.full((bsz, bc - pad0), -jnp.inf,
                                            jnp.float32)

    acc = acc_ref[...]
    acck = acck_ref[...]
    for k in range(nchunks):
        off = k * cw
        colc = c0 + off
        chunk = logits_ref[:, off:off + cw]
        x1 = jax.lax.bitcast_convert_type(base + (colc + np.int32(42)), jnp.uint32)
        bits = _threefry_bits(x1)
        u = jax.lax.bitcast_convert_type(
            (bits >> np.uint32(9)) | np.uint32(0x3F800000), jnp.float32) - 1.0
        w = -jnp.log(u)
        pert = chunk - jnp.log(w)
        acck = jnp.where(pert > acc, jnp.int32(j * nchunks + k), acck)
        acc = jnp.maximum(acc, pert)
    acc_ref[...] = acc
    acck_ref[...] = acck

    @pl.when(j == nblocks - 1)
    def _():
        col = acck * cw + lane
        m = jnp.max(acc, axis=1, keepdims=True)
        out_ref[...] = jnp.min(
            jnp.where(acc == m, col, _I32MAX), axis=1, keepdims=True)


def _build(shape, bc):
    bsz, vocab = shape
    nblocks = pl.cdiv(vocab, bc)
    return pl.pallas_call(
        partial(_sampler_kernel, bc=bc, vocab=vocab, nblocks=nblocks),
        grid=(nblocks,),
        in_specs=[pl.BlockSpec((bsz, bc), lambda j: (0, j))],
        out_specs=pl.BlockSpec((bsz, 1), lambda j: (0, 0)),
        out_shape=jax.ShapeDtypeStruct((bsz, 1), jnp.int32),
        scratch_shapes=[pltpu.VMEM((bsz, 512), jnp.float32),
                        pltpu.VMEM((bsz, 512), jnp.int32)],
    )


def kernel(logits):
    samples = _build(logits.shape, bc=4096)(logits)
    return samples.astype(jnp.int64)
